# Initial kernel scaffold; baseline (speedup 1.0000x reference)
import jax
import jax.numpy as jnp
from jax.experimental import pallas as pl


def kernel(X_input, tgt_table, pos_table):
    raise NotImplementedError("write your pallas kernel here")


# speedup vs baseline: 7.6322x; 7.6322x over previous
import jax
import jax.numpy as jnp
from jax import lax
from jax.experimental import pallas as pl
from jax.experimental.pallas import tpu as pltpu
from jax.experimental.pallas import tpu_sc as plsc

BATCH = 1024
SEQ = 200
D = 128
LANES = 16
NUM_WORKERS = 32
SEQ_PER_W = BATCH // NUM_WORKERS


def _emb_body(x_hbm, tgt_hbm, pos_hbm, out_hbm, idx_v, rows_v, pos_v, sem):
    info = plsc.get_sparse_core_info()
    wid = lax.axis_index("s") * info.num_cores + lax.axis_index("c")

    pltpu.sync_copy(pos_hbm, pos_v)

    def per_row(s, carry):
        b = wid * SEQ_PER_W + s
        pltpu.sync_copy(x_hbm.at[b], idx_v)
        cp1 = pltpu.async_copy(
            tgt_hbm.at[idx_v.at[pl.ds(0, 104)]], rows_v.at[pl.ds(0, 104)], sem
        )
        cp2 = pltpu.async_copy(
            tgt_hbm.at[idx_v.at[pl.ds(104, 96)]], rows_v.at[pl.ds(104, 96)], sem
        )
        cp1.wait()
        cp2.wait()

        def add_row(r, c2):
            for c in range(D // LANES):
                sl = pl.ds(c * LANES, LANES)
                rows_v[r, sl] = rows_v[r, sl] + pos_v[r, sl]
            return c2

        lax.fori_loop(0, SEQ, add_row, 0)
        pltpu.sync_copy(rows_v, out_hbm.at[b])
        return carry

    lax.fori_loop(0, SEQ_PER_W, per_row, 0)


def kernel(X_input, tgt_table, pos_table):
    mesh = plsc.VectorSubcoreMesh(core_axis_name="c", subcore_axis_name="s")
    run = pl.kernel(
        _emb_body,
        out_type=jax.ShapeDtypeStruct((BATCH, SEQ, D), jnp.float32),
        mesh=mesh,
        scratch_types=[
            pltpu.VMEM((SEQ,), jnp.int32),
            pltpu.VMEM((SEQ, D), jnp.float32),
            pltpu.VMEM((SEQ, D), jnp.float32),
            pltpu.SemaphoreType.DMA,
        ],
    )
    return run(X_input, tgt_table, pos_table)


# speedup vs baseline: 12.1944x; 1.5978x over previous
import jax
import jax.numpy as jnp
from jax import lax
from jax.experimental import pallas as pl
from jax.experimental.pallas import tpu as pltpu
from jax.experimental.pallas import tpu_sc as plsc

BATCH = 1024
SEQ = 200
D = 128
LANES = 16
NUM_WORKERS = 32
SEQ_PER_W = BATCH // NUM_WORKERS
SPLIT = 104


def _issue_gather(tgt_hbm, idx_all, s, rows, sem):
    pltpu.async_copy(
        tgt_hbm.at[idx_all.at[pl.ds(s * SEQ, SPLIT)]], rows.at[pl.ds(0, SPLIT)], sem
    )
    pltpu.async_copy(
        tgt_hbm.at[idx_all.at[pl.ds(s * SEQ + SPLIT, SEQ - SPLIT)]],
        rows.at[pl.ds(SPLIT, SEQ - SPLIT)],
        sem,
    )


def _emb_body(
    x_hbm, tgt_hbm, pos_hbm, out_hbm,
    idx_all, rows0, rows1, pos_v, gsem0, gsem1, wsem0, wsem1,
):
    info = plsc.get_sparse_core_info()
    wid = lax.axis_index("s") * info.num_cores + lax.axis_index("c")
    base = wid * SEQ_PER_W

    pltpu.sync_copy(pos_hbm, pos_v)
    pltpu.sync_copy(x_hbm.at[pl.ds(base * SEQ, SEQ_PER_W * SEQ)], idx_all)

    rows = (rows0, rows1)
    gsem = (gsem0, gsem1)
    wsem = (wsem0, wsem1)

    def drain_gather(p):
        pltpu.make_async_copy(out_hbm.at[0], rows[p], gsem[p]).wait()

    def drain_write(p):
        pltpu.make_async_copy(rows[p], out_hbm.at[0], wsem[p]).wait()

    _issue_gather(tgt_hbm, idx_all, 0, rows[0], gsem[0])

    @pl.loop(0, SEQ_PER_W, step=2)
    def per_pair(s):
        for p in range(2):
            s_eff = s + p

            @pl.when(s_eff >= 1)
            def _():
                drain_write(1 - p)

            @pl.when(s_eff + 1 < SEQ_PER_W)
            def _():
                _issue_gather(tgt_hbm, idx_all, s_eff + 1, rows[1 - p], gsem[1 - p])

            drain_gather(p)

            def add_row(r, carry):
                for c in range(D // LANES):
                    sl = pl.ds(c * LANES, LANES)
                    plsc.addupdate(rows[p].at[r, sl], pos_v[r, sl])
                return carry

            lax.fori_loop(0, SEQ, add_row, 0)
            pltpu.async_copy(rows[p], out_hbm.at[base + s_eff], wsem[p])

    drain_write(1)


def kernel(X_input, tgt_table, pos_table):
    mesh = plsc.VectorSubcoreMesh(core_axis_name="c", subcore_axis_name="s")
    run = pl.kernel(
        _emb_body,
        out_type=jax.ShapeDtypeStruct((BATCH, SEQ, D), jnp.float32),
        mesh=mesh,
        scratch_types=[
            pltpu.VMEM((SEQ_PER_W * SEQ,), jnp.int32),
            pltpu.VMEM((SEQ, D), jnp.float32),
            pltpu.VMEM((SEQ, D), jnp.float32),
            pltpu.VMEM((SEQ, D), jnp.float32),
            pltpu.SemaphoreType.DMA,
            pltpu.SemaphoreType.DMA,
            pltpu.SemaphoreType.DMA,
            pltpu.SemaphoreType.DMA,
        ],
    )
    return run(X_input.reshape(-1), tgt_table, pos_table)


# speedup vs baseline: 12.2341x; 1.0033x over previous
import jax
import jax.numpy as jnp
from jax import lax
from jax.experimental import pallas as pl
from jax.experimental.pallas import tpu as pltpu
from jax.experimental.pallas import tpu_sc as plsc

BATCH = 1024
SEQ = 200
D = 128
LANES = 16
NUM_WORKERS = 32
SEQ_PER_W = BATCH // NUM_WORKERS
SPLIT = 104


def _issue_gather(tgt_hbm, idx_all, s, rows, sem):
    pltpu.async_copy(
        tgt_hbm.at[idx_all.at[pl.ds(s * SEQ, SPLIT)]], rows.at[pl.ds(0, SPLIT)], sem
    )
    pltpu.async_copy(
        tgt_hbm.at[idx_all.at[pl.ds(s * SEQ + SPLIT, SEQ - SPLIT)]],
        rows.at[pl.ds(SPLIT, SEQ - SPLIT)],
        sem,
    )


def _emb_body(
    x_hbm, tgt_hbm, pos_hbm, out_hbm,
    idx_all, rows0, rows1, pos_v, gsem0, gsem1, wsem0, wsem1,
):
    info = plsc.get_sparse_core_info()
    wid = lax.axis_index("s") * info.num_cores + lax.axis_index("c")
    base = wid * SEQ_PER_W

    pltpu.sync_copy(pos_hbm, pos_v)
    pltpu.sync_copy(x_hbm.at[pl.ds(base * SEQ, SEQ_PER_W * SEQ)], idx_all)

    rows = (rows0, rows1)
    gsem = (gsem0, gsem1)
    wsem = (wsem0, wsem1)

    def drain_gather(p):
        pltpu.make_async_copy(out_hbm.at[0], rows[p], gsem[p]).wait()

    def drain_write(p):
        pltpu.make_async_copy(rows[p], out_hbm.at[0], wsem[p]).wait()

    _issue_gather(tgt_hbm, idx_all, 0, rows[0], gsem[0])

    @pl.loop(0, SEQ_PER_W, step=2)
    def per_pair(s):
        for p in range(2):
            s_eff = s + p

            @pl.when(s_eff >= 1)
            def _():
                drain_write(1 - p)

            @pl.when(s_eff + 1 < SEQ_PER_W)
            def _():
                _issue_gather(tgt_hbm, idx_all, s_eff + 1, rows[1 - p], gsem[1 - p])

            drain_gather(p)

            @plsc.parallel_loop(0, SEQ, 1, unroll=4)
            def add_row(r):
                for c in range(D // LANES):
                    sl = pl.ds(c * LANES, LANES)
                    plsc.addupdate(rows[p].at[r, sl], pos_v[r, sl])
            pltpu.async_copy(rows[p], out_hbm.at[base + s_eff], wsem[p])

    drain_write(1)


def kernel(X_input, tgt_table, pos_table):
    mesh = plsc.VectorSubcoreMesh(core_axis_name="c", subcore_axis_name="s")
    run = pl.kernel(
        _emb_body,
        out_type=jax.ShapeDtypeStruct((BATCH, SEQ, D), jnp.float32),
        mesh=mesh,
        scratch_types=[
            pltpu.VMEM((SEQ_PER_W * SEQ,), jnp.int32),
            pltpu.VMEM((SEQ, D), jnp.float32),
            pltpu.VMEM((SEQ, D), jnp.float32),
            pltpu.VMEM((SEQ, D), jnp.float32),
            pltpu.SemaphoreType.DMA,
            pltpu.SemaphoreType.DMA,
            pltpu.SemaphoreType.DMA,
            pltpu.SemaphoreType.DMA,
        ],
    )
    return run(X_input.reshape(-1), tgt_table, pos_table)


# speedup vs baseline: 12.2415x; 1.0006x over previous
import jax
import jax.numpy as jnp
from jax import lax
from jax.experimental import pallas as pl
from jax.experimental.pallas import tpu as pltpu
from jax.experimental.pallas import tpu_sc as plsc

BATCH = 1024
SEQ = 200
D = 128
LANES = 16
NUM_WORKERS = 32
SEQ_PER_W = BATCH // NUM_WORKERS
SPLIT = 104
NBUF = 3


def _issue_gather(tgt_hbm, idx_all, s, rows, sem):
    pltpu.async_copy(
        tgt_hbm.at[idx_all.at[pl.ds(s * SEQ, SPLIT)]], rows.at[pl.ds(0, SPLIT)], sem
    )
    pltpu.async_copy(
        tgt_hbm.at[idx_all.at[pl.ds(s * SEQ + SPLIT, SEQ - SPLIT)]],
        rows.at[pl.ds(SPLIT, SEQ - SPLIT)],
        sem,
    )


def _emb_body(
    x_hbm, tgt_hbm, pos_hbm, out_hbm,
    idx_all, rows0, rows1, rows2, pos_v, gsem0, gsem1, gsem2, wsem0, wsem1, wsem2,
):
    info = plsc.get_sparse_core_info()
    wid = lax.axis_index("s") * info.num_cores + lax.axis_index("c")
    base = wid * SEQ_PER_W

    pltpu.sync_copy(pos_hbm, pos_v)
    pltpu.sync_copy(x_hbm.at[pl.ds(base * SEQ, SEQ_PER_W * SEQ)], idx_all)

    rows = (rows0, rows1, rows2)
    gsem = (gsem0, gsem1, gsem2)
    wsem = (wsem0, wsem1, wsem2)

    def drain_gather(p):
        pltpu.make_async_copy(out_hbm.at[0], rows[p], gsem[p]).wait()

    def drain_write(p):
        pltpu.make_async_copy(rows[p], out_hbm.at[0], wsem[p]).wait()

    def add_and_write(p, s_eff):
        drain_gather(p)

        @plsc.parallel_loop(0, SEQ, 1, unroll=4)
        def add_row(r):
            for c in range(D // LANES):
                sl = pl.ds(c * LANES, LANES)
                plsc.addupdate(rows[p].at[r, sl], pos_v[r, sl])

        pltpu.async_copy(rows[p], out_hbm.at[base + s_eff], wsem[p])

    _issue_gather(tgt_hbm, idx_all, 0, rows[0], gsem[0])

    @pl.loop(0, SEQ_PER_W - 2, step=NBUF)
    def per_triple(s):
        for p in range(NBUF):
            s_eff = s + p

            @pl.when(s_eff >= 1)
            def _():
                drain_write((p + 2) % NBUF)

            _issue_gather(
                tgt_hbm, idx_all, s_eff + 1, rows[(p + 1) % NBUF], gsem[(p + 1) % NBUF]
            )
            add_and_write(p, s_eff)

    drain_write(2)
    _issue_gather(tgt_hbm, idx_all, SEQ_PER_W - 1, rows[1], gsem[1])
    add_and_write(0, SEQ_PER_W - 2)
    add_and_write(1, SEQ_PER_W - 1)
    drain_write(0)
    drain_write(1)


def kernel(X_input, tgt_table, pos_table):
    mesh = plsc.VectorSubcoreMesh(core_axis_name="c", subcore_axis_name="s")
    run = pl.kernel(
        _emb_body,
        out_type=jax.ShapeDtypeStruct((BATCH, SEQ, D), jnp.float32),
        mesh=mesh,
        scratch_types=[
            pltpu.VMEM((SEQ_PER_W * SEQ,), jnp.int32),
            pltpu.VMEM((SEQ, D), jnp.float32),
            pltpu.VMEM((SEQ, D), jnp.float32),
            pltpu.VMEM((SEQ, D), jnp.float32),
            pltpu.VMEM((SEQ, D), jnp.float32),
            pltpu.SemaphoreType.DMA,
            pltpu.SemaphoreType.DMA,
            pltpu.SemaphoreType.DMA,
            pltpu.SemaphoreType.DMA,
            pltpu.SemaphoreType.DMA,
            pltpu.SemaphoreType.DMA,
        ],
    )
    return run(X_input.reshape(-1), tgt_table, pos_table)


# speedup vs baseline: 13.0497x; 1.0660x over previous
import jax
import jax.numpy as jnp
from jax import lax
from jax.experimental import pallas as pl
from jax.experimental.pallas import tpu as pltpu
from jax.experimental.pallas import tpu_sc as plsc

BATCH = 1024
SEQ = 200
D = 128
LANES = 16
NUM_WORKERS = 32
SEQ_PER_W = BATCH // NUM_WORKERS
SPLIT = 104
NBUF = 3


def _issue_gather(tgt_hbm, idx_all, s, rows, sem):
    pltpu.async_copy(
        tgt_hbm.at[idx_all.at[pl.ds(s * SEQ, SPLIT)]], rows.at[pl.ds(0, SPLIT)], sem
    )
    pltpu.async_copy(
        tgt_hbm.at[idx_all.at[pl.ds(s * SEQ + SPLIT, SEQ - SPLIT)]],
        rows.at[pl.ds(SPLIT, SEQ - SPLIT)],
        sem,
    )


def _emb_body(
    x_hbm, tgt_hbm, pos_hbm, out_hbm,
    idx_all, rows0, rows1, rows2, pos_v, gsem0, gsem1, gsem2, wsem0, wsem1, wsem2,
):
    info = plsc.get_sparse_core_info()
    wid = lax.axis_index("s") * info.num_cores + lax.axis_index("c")
    base = wid * SEQ_PER_W

    pltpu.sync_copy(pos_hbm, pos_v)
    pltpu.sync_copy(x_hbm.at[pl.ds(base * SEQ, SEQ_PER_W * SEQ)], idx_all)

    rows = (rows0, rows1, rows2)
    gsem = (gsem0, gsem1, gsem2)
    wsem = (wsem0, wsem1, wsem2)

    def drain_gather(p):
        pltpu.make_async_copy(out_hbm.at[0], rows[p], gsem[p]).wait()

    def drain_write(p):
        pltpu.make_async_copy(rows[p], out_hbm.at[0], wsem[p]).wait()

    def add_and_write(p, s_eff):
        drain_gather(p)

        @plsc.parallel_loop(0, SPLIT, 1, unroll=8)
        def add_row_lo(r):
            for c in range(D // LANES):
                sl = pl.ds(c * LANES, LANES)
                plsc.addupdate(rows[p].at[r, sl], pos_v[r, sl])

        pltpu.async_copy(
            rows[p].at[pl.ds(0, SPLIT)],
            out_hbm.at[base + s_eff].at[pl.ds(0, SPLIT)],
            wsem[p],
        )

        @plsc.parallel_loop(SPLIT, SEQ, 1, unroll=8)
        def add_row_hi(r):
            for c in range(D // LANES):
                sl = pl.ds(c * LANES, LANES)
                plsc.addupdate(rows[p].at[r, sl], pos_v[r, sl])

        pltpu.async_copy(
            rows[p].at[pl.ds(SPLIT, SEQ - SPLIT)],
            out_hbm.at[base + s_eff].at[pl.ds(SPLIT, SEQ - SPLIT)],
            wsem[p],
        )

    _issue_gather(tgt_hbm, idx_all, 0, rows[0], gsem[0])

    @pl.loop(0, SEQ_PER_W - 2, step=NBUF)
    def per_triple(s):
        for p in range(NBUF):
            s_eff = s + p

            @pl.when(s_eff >= 1)
            def _():
                drain_write((p + 2) % NBUF)

            _issue_gather(
                tgt_hbm, idx_all, s_eff + 1, rows[(p + 1) % NBUF], gsem[(p + 1) % NBUF]
            )
            add_and_write(p, s_eff)

    drain_write(2)
    _issue_gather(tgt_hbm, idx_all, SEQ_PER_W - 1, rows[1], gsem[1])
    add_and_write(0, SEQ_PER_W - 2)
    add_and_write(1, SEQ_PER_W - 1)
    drain_write(0)
    drain_write(1)


def kernel(X_input, tgt_table, pos_table):
    mesh = plsc.VectorSubcoreMesh(core_axis_name="c", subcore_axis_name="s")
    run = pl.kernel(
        _emb_body,
        out_type=jax.ShapeDtypeStruct((BATCH, SEQ, D), jnp.float32),
        mesh=mesh,
        scratch_types=[
            pltpu.VMEM((SEQ_PER_W * SEQ,), jnp.int32),
            pltpu.VMEM((SEQ, D), jnp.float32),
            pltpu.VMEM((SEQ, D), jnp.float32),
            pltpu.VMEM((SEQ, D), jnp.float32),
            pltpu.VMEM((SEQ, D), jnp.float32),
            pltpu.SemaphoreType.DMA,
            pltpu.SemaphoreType.DMA,
            pltpu.SemaphoreType.DMA,
            pltpu.SemaphoreType.DMA,
            pltpu.SemaphoreType.DMA,
            pltpu.SemaphoreType.DMA,
        ],
    )
    return run(X_input.reshape(-1), tgt_table, pos_table)
